# speedup vs baseline: 1.4541x; 1.4541x over previous
import functools

import jax
import jax.numpy as jnp
from jax import lax
from jax.experimental import pallas as pl
from jax.experimental.pallas import tpu as pltpu
from jax.experimental.pallas import tpu_sc as plsc

D_MODEL = 1024
N_ROWS = 32768
NC = 2
NS = 16
NW = NC * NS
PER_W = N_ROWS // NW
CHUNK = 64
NCHUNK = PER_W // CHUNK


def _sc_gather(table, idx):
    mesh = plsc.VectorSubcoreMesh(core_axis_name="c", subcore_axis_name="s")

    @functools.partial(
        pl.kernel,
        mesh=mesh,
        out_type=jax.ShapeDtypeStruct((N_ROWS, D_MODEL), jnp.float32),
        scratch_types=[
            pltpu.VMEM((PER_W,), jnp.int32),
            pltpu.VMEM((CHUNK, D_MODEL), jnp.float32),
            pltpu.SemaphoreType.DMA,
        ],
    )
    def k(table_hbm, idx_hbm, out_hbm, idx_v, buf, sem):
        wid = lax.axis_index("s") * NC + lax.axis_index("c")
        base = wid * PER_W
        pltpu.sync_copy(idx_hbm.at[pl.ds(base, PER_W)], idx_v)
        for i in range(NCHUNK):
            pltpu.async_copy(
                table_hbm.at[idx_v.at[pl.ds(i * CHUNK, CHUNK)]], buf, sem
            ).wait()
            pltpu.sync_copy(buf, out_hbm.at[pl.ds(base + i * CHUNK, CHUNK)])

    return k(table, idx)


def kernel(positions, pe):
    idx = positions.reshape(-1).astype(jnp.int32)
    table = pe.reshape(pe.shape[-2], pe.shape[-1])
    out = _sc_gather(table, idx)
    return out.reshape(positions.shape[0], positions.shape[1], D_MODEL)


# speedup vs baseline: 1.5758x; 1.0837x over previous
import functools

import jax
import jax.numpy as jnp
from jax import lax
from jax.experimental import pallas as pl
from jax.experimental.pallas import tpu as pltpu
from jax.experimental.pallas import tpu_sc as plsc

D_MODEL = 1024
N_ROWS = 32768
NC = 2
NS = 16
NW = NC * NS
PER_W = N_ROWS // NW
CHUNK = 32
NCHUNK = PER_W // CHUNK


def _sc_gather(table, idx):
    mesh = plsc.VectorSubcoreMesh(core_axis_name="c", subcore_axis_name="s")

    @functools.partial(
        pl.kernel,
        mesh=mesh,
        out_type=jax.ShapeDtypeStruct((N_ROWS, D_MODEL), jnp.float32),
        scratch_types=[
            pltpu.VMEM((PER_W,), jnp.int32),
            pltpu.VMEM((2, CHUNK, D_MODEL), jnp.float32),
            pltpu.SemaphoreType.DMA((2,)),
            pltpu.SemaphoreType.DMA((2,)),
        ],
    )
    def k(table_hbm, idx_hbm, out_hbm, idx_v, bufs, gsem, psem):
        wid = lax.axis_index("s") * NC + lax.axis_index("c")
        base = wid * PER_W
        pltpu.sync_copy(idx_hbm.at[pl.ds(base, PER_W)], idx_v)

        def g_start(i, b):
            pltpu.async_copy(
                table_hbm.at[idx_v.at[pl.ds(i * CHUNK, CHUNK)]],
                bufs.at[b], gsem.at[b])

        def g_wait(i, b):
            pltpu.make_async_copy(
                table_hbm.at[idx_v.at[pl.ds(i * CHUNK, CHUNK)]],
                bufs.at[b], gsem.at[b]).wait()

        def p_start(i, b):
            pltpu.async_copy(
                bufs.at[b], out_hbm.at[pl.ds(base + i * CHUNK, CHUNK)],
                psem.at[b])

        def p_wait(i, b):
            pltpu.make_async_copy(
                bufs.at[b], out_hbm.at[pl.ds(base + i * CHUNK, CHUNK)],
                psem.at[b]).wait()

        g_start(0, 0)
        g_wait(0, 0)
        p_start(0, 0)
        g_start(1, 1)

        def body(g, _):
            for u in range(2):
                i = 2 * g + 1 + u
                b = (1 + u) % 2
                g_wait(i, b)
                p_start(i, b)
                p_wait(i - 1, 1 - b)
                g_start(i + 1, 1 - b)
            return 0

        lax.fori_loop(0, (NCHUNK - 2) // 2, body, 0)

        g_wait(NCHUNK - 1, 1)
        p_start(NCHUNK - 1, 1)
        p_wait(NCHUNK - 2, 0)
        p_wait(NCHUNK - 1, 1)

    return k(table, idx)


def kernel(positions, pe):
    idx = positions.reshape(-1).astype(jnp.int32)
    table = pe.reshape(pe.shape[-2], pe.shape[-1])
    out = _sc_gather(table, idx)
    return out.reshape(positions.shape[0], positions.shape[1], D_MODEL)


# speedup vs baseline: 1.6141x; 1.0243x over previous
import functools

import jax
import jax.numpy as jnp
from jax import lax
from jax.experimental import pallas as pl
from jax.experimental.pallas import tpu as pltpu
from jax.experimental.pallas import tpu_sc as plsc

D_MODEL = 1024
N_ROWS = 32768
NC = 2
NS = 16
NW = NC * NS
PER_W = N_ROWS // NW
CHUNK = 32
NCHUNK = PER_W // CHUNK


def _sc_gather(table, idx):
    mesh = plsc.VectorSubcoreMesh(core_axis_name="c", subcore_axis_name="s")

    @functools.partial(
        pl.kernel,
        mesh=mesh,
        out_type=jax.ShapeDtypeStruct((N_ROWS, D_MODEL), jnp.float32),
        scratch_types=[
            pltpu.VMEM((PER_W,), jnp.int32),
            pltpu.VMEM((3, CHUNK, D_MODEL), jnp.float32),
            pltpu.SemaphoreType.DMA((3,)),
            pltpu.SemaphoreType.DMA((3,)),
        ],
    )
    def k(table_hbm, idx_hbm, out_hbm, idx_v, bufs, gsem, psem):
        wid = lax.axis_index("s") * NC + lax.axis_index("c")
        base = wid * PER_W
        pltpu.sync_copy(idx_hbm.at[pl.ds(base, PER_W)], idx_v)

        def g_start(i, b):
            pltpu.async_copy(
                table_hbm.at[idx_v.at[pl.ds(i * CHUNK, CHUNK)]],
                bufs.at[b], gsem.at[b])

        def g_wait(i, b):
            pltpu.make_async_copy(
                table_hbm.at[idx_v.at[pl.ds(i * CHUNK, CHUNK)]],
                bufs.at[b], gsem.at[b]).wait()

        def p_start(i, b):
            pltpu.async_copy(
                bufs.at[b], out_hbm.at[pl.ds(base + i * CHUNK, CHUNK)],
                psem.at[b])

        def p_wait(i, b):
            pltpu.make_async_copy(
                bufs.at[b], out_hbm.at[pl.ds(base + i * CHUNK, CHUNK)],
                psem.at[b]).wait()

        g_start(0, 0)
        g_start(1, 1)

        g_wait(0, 0)
        p_start(0, 0)
        g_start(2, 2)
        g_wait(1, 1)
        p_start(1, 1)
        p_wait(0, 0)
        g_start(3, 0)

        def body(g, _):
            for u in range(3):
                i = 3 * g + 2 + u
                b = (2 + u) % 3
                g_wait(i, b)
                p_start(i, b)
                p_wait(i - 1, (b + 2) % 3)
                g_start(i + 2, (b + 2) % 3)
            return 0

        lax.fori_loop(0, (NCHUNK - 5) // 3, body, 0)

        g_wait(NCHUNK - 3, 2)
        p_start(NCHUNK - 3, 2)
        p_wait(NCHUNK - 4, 1)
        g_start(NCHUNK - 1, 1)
        g_wait(NCHUNK - 2, 0)
        p_start(NCHUNK - 2, 0)
        p_wait(NCHUNK - 3, 2)
        g_wait(NCHUNK - 1, 1)
        p_start(NCHUNK - 1, 1)
        p_wait(NCHUNK - 2, 0)
        p_wait(NCHUNK - 1, 1)

    return k(table, idx)


def kernel(positions, pe):
    idx = positions.reshape(-1).astype(jnp.int32)
    table = pe.reshape(pe.shape[-2], pe.shape[-1])
    out = _sc_gather(table, idx)
    return out.reshape(positions.shape[0], positions.shape[1], D_MODEL)


# speedup vs baseline: 1.6240x; 1.0061x over previous
import functools

import jax
import jax.numpy as jnp
from jax import lax
from jax.experimental import pallas as pl
from jax.experimental.pallas import tpu as pltpu
from jax.experimental.pallas import tpu_sc as plsc

D_MODEL = 1024
N_ROWS = 32768
NC = 2
NS = 16
NW = NC * NS
PER_W = N_ROWS // NW
CHUNK = 16
NCHUNK = PER_W // CHUNK
NBUF = 4
LOOK = NBUF - 1


def _sc_gather(table, idx):
    mesh = plsc.VectorSubcoreMesh(core_axis_name="c", subcore_axis_name="s")

    @functools.partial(
        pl.kernel,
        mesh=mesh,
        out_type=jax.ShapeDtypeStruct((N_ROWS, D_MODEL), jnp.float32),
        scratch_types=[
            pltpu.VMEM((PER_W,), jnp.int32),
            pltpu.VMEM((NBUF, CHUNK, D_MODEL), jnp.float32),
            pltpu.SemaphoreType.DMA((NBUF,)),
            pltpu.SemaphoreType.DMA((NBUF,)),
        ],
    )
    def k(table_hbm, idx_hbm, out_hbm, idx_v, bufs, gsem, psem):
        wid = lax.axis_index("s") * NC + lax.axis_index("c")
        base = wid * PER_W
        pltpu.sync_copy(idx_hbm.at[pl.ds(base, PER_W)], idx_v)

        def g_start(i, b):
            pltpu.async_copy(
                table_hbm.at[idx_v.at[pl.ds(i * CHUNK, CHUNK)]],
                bufs.at[b], gsem.at[b])

        def g_wait(i, b):
            pltpu.make_async_copy(
                table_hbm.at[idx_v.at[pl.ds(i * CHUNK, CHUNK)]],
                bufs.at[b], gsem.at[b]).wait()

        def p_start(i, b):
            pltpu.async_copy(
                bufs.at[b], out_hbm.at[pl.ds(base + i * CHUNK, CHUNK)],
                psem.at[b])

        def p_wait(i, b):
            pltpu.make_async_copy(
                bufs.at[b], out_hbm.at[pl.ds(base + i * CHUNK, CHUNK)],
                psem.at[b]).wait()

        def step(i, b, do_pwait, do_refill):
            g_wait(i, b)
            p_start(i, b)
            if do_pwait:
                p_wait(i - 1, (i - 1) % NBUF)
            if do_refill:
                g_start(i + LOOK, (i + LOOK) % NBUF)

        for j in range(LOOK):
            g_start(j, j)

        for i in range(NBUF):
            step(i, i, i >= 1, i + LOOK < NCHUNK)

        assert NCHUNK % NBUF == 0 and NCHUNK >= 3 * NBUF

        def body(g, _):
            for u in range(NBUF):
                i = NBUF * (g + 1) + u
                step(i, u, True, True)
            return 0

        lax.fori_loop(0, (NCHUNK - 2 * NBUF) // NBUF, body, 0)

        for i in range(NCHUNK - NBUF, NCHUNK):
            step(i, i % NBUF, True, i + LOOK < NCHUNK)
        p_wait(NCHUNK - 1, (NCHUNK - 1) % NBUF)

    return k(table, idx)


def kernel(positions, pe):
    idx = positions.reshape(-1).astype(jnp.int32)
    table = pe.reshape(pe.shape[-2], pe.shape[-1])
    out = _sc_gather(table, idx)
    return out.reshape(positions.shape[0], positions.shape[1], D_MODEL)
